# Initial kernel scaffold; baseline (speedup 1.0000x reference)
import jax
import jax.numpy as jnp
from jax.experimental import pallas as pl


def kernel(inputs, emb_table, pos_table):
    raise NotImplementedError("write your pallas kernel here")


# speedup vs baseline: 7.0470x; 7.0470x over previous
import functools

import jax
import jax.numpy as jnp
from jax import lax
from jax.experimental import pallas as pl
from jax.experimental.pallas import tpu as pltpu
from jax.experimental.pallas import tpu_sc as plsc


def _sc_embed(seq_len: int, d_model: int, n_rows: int):
    info = plsc.get_sparse_core_info()
    nc, ns, lanes = info.num_cores, info.num_subcores, info.num_lanes
    nw = nc * ns

    assert n_rows % (nw * seq_len) == 0
    seqs_per_w = n_rows // (nw * seq_len)
    assert d_model % lanes == 0
    d_vecs = d_model // lanes

    assert seq_len % 8 == 0 and seq_len <= 256
    split_a = min(104, seq_len)
    split_b = seq_len - split_a

    NBUF = 4
    assert seqs_per_w % NBUF == 0

    mesh = plsc.VectorSubcoreMesh(core_axis_name="c", subcore_axis_name="s")

    @functools.partial(
        pl.kernel,
        mesh=mesh,
        out_type=jax.ShapeDtypeStruct((n_rows, d_model), jnp.float32),
        scratch_types=[
            pltpu.VMEM((NBUF, 2, split_a), jnp.int32),
            pltpu.VMEM((NBUF, seq_len, d_model), jnp.float32),
            pltpu.VMEM((seq_len, d_model), jnp.float32),
        ]
        + [pltpu.SemaphoreType.DMA] * (2 * NBUF),
    )
    def k(idx_hbm, emb_hbm, pos_hbm, out_hbm, idx_v, rows_v, pos_v, *sems):
        sem_g = sems[:NBUF]
        sem_s = sems[NBUF:]
        wid = lax.axis_index("s") * nc + lax.axis_index("c")
        wbase = wid * (seqs_per_w * seq_len)

        pltpu.sync_copy(pos_hbm, pos_v)

        def load_idx_and_gather(g, b):
            base = wbase + g * seq_len
            pltpu.sync_copy(idx_hbm.at[pl.ds(base, split_a)], idx_v.at[b, 0])
            pltpu.async_copy(
                emb_hbm.at[idx_v.at[b, 0]],
                rows_v.at[b, pl.ds(0, split_a)],
                sem_g[b],
            )
            if split_b:
                pltpu.sync_copy(
                    idx_hbm.at[pl.ds(base + split_a, split_b)],
                    idx_v.at[b, 1, pl.ds(0, split_b)],
                )
                pltpu.async_copy(
                    emb_hbm.at[idx_v.at[b, 1, pl.ds(0, split_b)]],
                    rows_v.at[b, pl.ds(split_a, split_b)],
                    sem_g[b],
                )

        def wait_gather(b):
            pltpu.make_async_copy(
                emb_hbm.at[idx_v.at[b, 0]],
                rows_v.at[b, pl.ds(0, split_a)],
                sem_g[b],
            ).wait()
            if split_b:
                pltpu.make_async_copy(
                    emb_hbm.at[idx_v.at[b, 1, pl.ds(0, split_b)]],
                    rows_v.at[b, pl.ds(split_a, split_b)],
                    sem_g[b],
                ).wait()

        def wait_scatter(b):
            pltpu.make_async_copy(
                rows_v.at[b], out_hbm.at[pl.ds(0, seq_len)], sem_s[b]
            ).wait()

        for j in range(NBUF):
            load_idx_and_gather(jnp.int32(j), j)

        @pl.loop(0, seqs_per_w, step=NBUF)
        def chunk_loop(i0):
            for j in range(NBUF):
                i = i0 + j
                b = j
                wait_gather(b)

                @pl.loop(0, seq_len)
                def add_row(r):
                    for c in range(d_vecs):
                        sl = pl.ds(c * lanes, lanes)
                        rows_v[b, r, sl] = rows_v[b, r, sl] + pos_v[r, sl]

                pltpu.async_copy(
                    rows_v.at[b],
                    out_hbm.at[pl.ds((wbase + i * seq_len), seq_len)],
                    sem_s[b],
                )

                bp = (j - 1) % NBUF
                g = i - 1 + NBUF

                @pl.when(jnp.logical_and(i >= 1, g < seqs_per_w))
                def prefetch():
                    wait_scatter(bp)
                    load_idx_and_gather(g, bp)

        for b in range(NBUF):
            wait_scatter(b)

    return k


def kernel(inputs, emb_table, pos_table):
    bsz, seq_len = inputs.shape
    d_model = emb_table.shape[1]
    n_rows = bsz * seq_len
    idx_flat = inputs.reshape(n_rows).astype(jnp.int32)
    pos = pos_table[:seq_len].astype(jnp.float32)
    run = _sc_embed(seq_len, d_model, n_rows)
    out_flat = run(idx_flat, emb_table.astype(jnp.float32), pos)
    return out_flat.reshape(bsz, seq_len, d_model)
